# baseline (speedup 1.0000x reference)
import dataclasses

import jax
import jax.numpy as jnp
from jax import lax
from jax.experimental import pallas as pl
from jax.experimental.pallas import tpu as pltpu
from jax.experimental.pallas import tpu_sc as plsc

N = 10000
E = 320000
D = 128
NC = 2
NS = 16
NW = NC * NS
FPT = D // NW
VL = 16
ECHUNK = 4000
DCHUNK = 2000
R = N
G = N // R

_vmesh = plsc.VectorSubcoreMesh(core_axis_name="c", subcore_axis_name="s")

_sc_params = pltpu.CompilerParams()
if "needs_layout_passes" in pltpu.CompilerParams.__dataclass_fields__:
    _sc_params = dataclasses.replace(_sc_params, needs_layout_passes=False)


def _deg_body(col_hbm, ew_hbm, out_hbm, col_v, ew_v, deg_v):
    wid = lax.axis_index("s") * NC + lax.axis_index("c")

    @pl.loop(0, N, step=VL)
    def _zero(i):
        deg_v[pl.ds(i, VL)] = jnp.zeros((VL,), jnp.float32)

    epw = E // NW
    base = wid * epw

    @pl.loop(0, epw, step=DCHUNK)
    def _chunk(i):
        pltpu.sync_copy(col_hbm.at[pl.ds(base + i, DCHUNK)], col_v)
        pltpu.sync_copy(ew_hbm.at[pl.ds(base + i, DCHUNK)], ew_v)

        @plsc.parallel_loop(0, DCHUNK, step=VL, unroll=4)
        def _vec(j):
            c = col_v[pl.ds(j, VL)]
            w = ew_v[pl.ds(j, VL)]
            plsc.addupdate_scatter(deg_v, [c], w)

    pltpu.sync_copy(deg_v, out_hbm.at[wid])


@jax.jit
def _deg_partials(col, ew):
    k = pl.kernel(
        _deg_body,
        out_type=jax.ShapeDtypeStruct((NW, N), jnp.float32),
        mesh=_vmesh,
        compiler_params=_sc_params,
        scratch_types=[
            pltpu.VMEM((DCHUNK,), jnp.int32),
            pltpu.VMEM((DCHUNK,), jnp.float32),
            pltpu.VMEM((N,), jnp.float32),
        ],
    )
    return k(col, ew)


def _agg_body(yT_hbm, row_hbm, col_hbm, ew_hbm, out_hbm,
              y_v, acc_v, row0, col0, ew0, row1, col1, ew1, sem0, sem1):
    wid = lax.axis_index("s") * NC + lax.axis_index("c")
    pltpu.sync_copy(yT_hbm.at[pl.ds(wid * FPT, FPT)], y_v)

    for f in range(FPT):
        @plsc.parallel_loop(0, N, step=VL, unroll=8)
        def _zero(i, f=f):
            acc_v[f, pl.ds(i, VL)] = jnp.zeros((VL,), jnp.float32)

    bufs = ((row0, col0, ew0, sem0), (row1, col1, ew1, sem1))
    nchunk = E // ECHUNK

    def start(slot, ci):
        r, c, w, s = bufs[slot]
        off = jnp.minimum(ci * ECHUNK, E - ECHUNK)
        pltpu.async_copy(row_hbm.at[pl.ds(off, ECHUNK)], r, s)
        pltpu.async_copy(col_hbm.at[pl.ds(off, ECHUNK)], c, s)
        pltpu.async_copy(ew_hbm.at[pl.ds(off, ECHUNK)], w, s)

    def wait(slot):
        r, c, w, s = bufs[slot]
        pltpu.make_async_copy(row_hbm.at[pl.ds(0, ECHUNK)], r, s).wait()
        pltpu.make_async_copy(col_hbm.at[pl.ds(0, ECHUNK)], c, s).wait()
        pltpu.make_async_copy(ew_hbm.at[pl.ds(0, ECHUNK)], w, s).wait()

    def process(slot):
        r, c, w, _ = bufs[slot]

        @plsc.parallel_loop(0, ECHUNK, step=VL, unroll=10)
        def _vec(j):
            rr = r[pl.ds(j, VL)]
            cc = c[pl.ds(j, VL)]
            ww = w[pl.ds(j, VL)]
            for f in range(FPT):
                fi = jnp.full((VL,), f, jnp.int32)
                vals = plsc.load_gather(y_v, [fi, rr])
                plsc.addupdate_scatter(acc_v, [fi, cc], vals * ww)

    start(0, 0)
    start(1, 1)

    @pl.loop(0, nchunk, step=2)
    def _outer(i):
        wait(0)
        process(0)
        start(0, i + 2)
        wait(1)
        process(1)
        start(1, i + 3)

    wait(0)
    wait(1)
    pltpu.sync_copy(acc_v, out_hbm.at[pl.ds(wid * FPT, FPT)])


@jax.jit
def _agg(yT, row, col, ew):
    k = pl.kernel(
        _agg_body,
        out_type=jax.ShapeDtypeStruct((D, N), jnp.float32),
        mesh=_vmesh,
        compiler_params=_sc_params,
        scratch_types=[
            pltpu.VMEM((FPT, N), jnp.float32),
            pltpu.VMEM((FPT, N), jnp.float32),
            pltpu.VMEM((ECHUNK,), jnp.int32),
            pltpu.VMEM((ECHUNK,), jnp.int32),
            pltpu.VMEM((ECHUNK,), jnp.float32),
            pltpu.VMEM((ECHUNK,), jnp.int32),
            pltpu.VMEM((ECHUNK,), jnp.int32),
            pltpu.VMEM((ECHUNK,), jnp.float32),
            pltpu.SemaphoreType.DMA,
            pltpu.SemaphoreType.DMA,
        ],
    )
    return k(yT, row, col, ew)


def _mm_t_body(W_ref, x_ref, o_ref):
    o_ref[...] = lax.dot_general(
        W_ref[...], x_ref[...], (((0,), (1,)), ((), ())),
        preferred_element_type=jnp.float32)


def _mm_t(W, x):
    return pl.pallas_call(
        _mm_t_body,
        grid=(G,),
        in_specs=[
            pl.BlockSpec((D, D), lambda i: (0, 0)),
            pl.BlockSpec((R, D), lambda i: (i, 0)),
        ],
        out_specs=pl.BlockSpec((D, R), lambda i: (0, i)),
        out_shape=jax.ShapeDtypeStruct((D, N), jnp.float32),
    )(W, x)


def _dis_y_body(degp_ref, xwT_ref, dis_ref, yT_ref):
    deg = jnp.sum(degp_ref[...], axis=0, keepdims=True) + 1.0
    dis = jnp.where(deg > 0, lax.rsqrt(deg), 0.0)
    dis_ref[...] = dis
    yT_ref[...] = xwT_ref[...] * dis


def _dis_y(deg_part, xwT):
    return pl.pallas_call(
        _dis_y_body,
        grid=(G,),
        in_specs=[
            pl.BlockSpec((NW, R), lambda i: (0, i)),
            pl.BlockSpec((D, R), lambda i: (0, i)),
        ],
        out_specs=[
            pl.BlockSpec((1, R), lambda i: (0, i)),
            pl.BlockSpec((D, R), lambda i: (0, i)),
        ],
        out_shape=[
            jax.ShapeDtypeStruct((1, N), jnp.float32),
            jax.ShapeDtypeStruct((D, N), jnp.float32),
        ],
    )(deg_part, xwT)


def _post_mm_body(aggT_ref, yT_ref, dis_ref, b_ref, a_ref, W_ref, yT_out_ref):
    dis = dis_ref[...]
    h = dis * (aggT_ref[...] + yT_ref[...]) + b_ref[...]
    h = jnp.where(h >= 0, h, a_ref[...] * h)
    xwT = lax.dot_general(
        W_ref[...], h, (((0,), (0,)), ((), ())),
        preferred_element_type=jnp.float32)
    yT_out_ref[...] = xwT * dis


def _post_mm(aggT, yT, dis2d, b, a, W):
    return pl.pallas_call(
        _post_mm_body,
        grid=(G,),
        in_specs=[
            pl.BlockSpec((D, R), lambda i: (0, i)),
            pl.BlockSpec((D, R), lambda i: (0, i)),
            pl.BlockSpec((1, R), lambda i: (0, i)),
            pl.BlockSpec((D, 1), lambda i: (0, 0)),
            pl.BlockSpec((D, 1), lambda i: (0, 0)),
            pl.BlockSpec((D, D), lambda i: (0, 0)),
        ],
        out_specs=pl.BlockSpec((D, R), lambda i: (0, i)),
        out_shape=jax.ShapeDtypeStruct((D, N), jnp.float32),
    )(aggT, yT, dis2d, b, a, W)


def _final_body(aggT_ref, yT_ref, dis_ref, b_ref, a_ref, o_ref):
    h = dis_ref[...] * (aggT_ref[...] + yT_ref[...]) + b_ref[...]
    h = jnp.where(h >= 0, h, a_ref[...] * h)
    o_ref[...] = h.T


def _final(aggT, yT, dis2d, b, a):
    return pl.pallas_call(
        _final_body,
        grid=(G,),
        in_specs=[
            pl.BlockSpec((D, R), lambda i: (0, i)),
            pl.BlockSpec((D, R), lambda i: (0, i)),
            pl.BlockSpec((1, R), lambda i: (0, i)),
            pl.BlockSpec((D, 1), lambda i: (0, 0)),
            pl.BlockSpec((D, 1), lambda i: (0, 0)),
        ],
        out_specs=pl.BlockSpec((R, D), lambda i: (i, 0)),
        out_shape=jax.ShapeDtypeStruct((N, D), jnp.float32),
    )(aggT, yT, dis2d, b, a)


def kernel(x, edge_index, edge_weight, W1, b1, a1, W2, b2, a2, W3, b3, a3):
    row = edge_index[0]
    col = edge_index[1]
    ew = edge_weight

    deg_part = _deg_partials(col, ew)
    xw1T = _mm_t(W1, x)
    dis2d, y1T = _dis_y(deg_part, xw1T)

    b1c, a1c = b1.reshape(D, 1), a1.reshape(D, 1)
    b2c, a2c = b2.reshape(D, 1), a2.reshape(D, 1)
    b3c, a3c = b3.reshape(D, 1), a3.reshape(D, 1)

    agg1T = _agg(y1T, row, col, ew)
    y2T = _post_mm(agg1T, y1T, dis2d, b1c, a1c, W2)
    agg2T = _agg(y2T, row, col, ew)
    y3T = _post_mm(agg2T, y2T, dis2d, b2c, a2c, W3)
    agg3T = _agg(y3T, row, col, ew)
    return _final(agg3T, y3T, dis2d, b3c, a3c)


# speedup vs baseline: 1.1195x; 1.1195x over previous
import dataclasses

import jax
import jax.numpy as jnp
from jax import lax
from jax.experimental import pallas as pl
from jax.experimental.pallas import tpu as pltpu
from jax.experimental.pallas import tpu_sc as plsc

N = 10000
E = 320000
D = 128
NC = 2
NS = 16
NW = NC * NS
FPT = D // NW
VL = 16
ECHUNK = 4000
DCHUNK = 2000
R = N
G = N // R

_vmesh = plsc.VectorSubcoreMesh(core_axis_name="c", subcore_axis_name="s")

_sc_params = pltpu.CompilerParams()
if "needs_layout_passes" in pltpu.CompilerParams.__dataclass_fields__:
    _sc_params = dataclasses.replace(_sc_params, needs_layout_passes=False)


def _deg_body(col_hbm, ew_hbm, out_hbm, col_v, ew_v, deg_v):
    wid = lax.axis_index("s") * NC + lax.axis_index("c")

    @pl.loop(0, N, step=VL)
    def _zero(i):
        deg_v[pl.ds(i, VL)] = jnp.zeros((VL,), jnp.float32)

    epw = E // NW
    base = wid * epw

    @pl.loop(0, epw, step=DCHUNK)
    def _chunk(i):
        pltpu.sync_copy(col_hbm.at[pl.ds(base + i, DCHUNK)], col_v)
        pltpu.sync_copy(ew_hbm.at[pl.ds(base + i, DCHUNK)], ew_v)

        @plsc.parallel_loop(0, DCHUNK, step=VL, unroll=4)
        def _vec(j):
            c = col_v[pl.ds(j, VL)]
            w = ew_v[pl.ds(j, VL)]
            plsc.addupdate_scatter(deg_v, [c], w)

    pltpu.sync_copy(deg_v, out_hbm.at[wid])


@jax.jit
def _deg_partials(col, ew):
    k = pl.kernel(
        _deg_body,
        out_type=jax.ShapeDtypeStruct((NW, N), jnp.float32),
        mesh=_vmesh,
        compiler_params=_sc_params,
        scratch_types=[
            pltpu.VMEM((DCHUNK,), jnp.int32),
            pltpu.VMEM((DCHUNK,), jnp.float32),
            pltpu.VMEM((N,), jnp.float32),
        ],
    )
    return k(col, ew)


def _agg_body(yT_hbm, pk_hbm, ew_hbm, out_hbm,
              y0, y1, y2, y3, a0, a1, a2, a3,
              pk0, ew0, pk1, ew1, sem0, sem1):
    wid = lax.axis_index("s") * NC + lax.axis_index("c")
    ys = (y0, y1, y2, y3)
    accs = (a0, a1, a2, a3)
    for f in range(FPT):
        pltpu.sync_copy(yT_hbm.at[wid * FPT + f], ys[f])

        @plsc.parallel_loop(0, N, step=VL, unroll=8)
        def _zero(i, f=f):
            accs[f][pl.ds(i, VL)] = jnp.zeros((VL,), jnp.float32)

    bufs = ((pk0, ew0, sem0), (pk1, ew1, sem1))
    nchunk = E // ECHUNK

    def start(slot, ci):
        p, w, s = bufs[slot]
        off = jnp.minimum(ci * ECHUNK, E - ECHUNK)
        pltpu.async_copy(pk_hbm.at[pl.ds(off, ECHUNK)], p, s)
        pltpu.async_copy(ew_hbm.at[pl.ds(off, ECHUNK)], w, s)

    def wait(slot):
        p, w, s = bufs[slot]
        pltpu.make_async_copy(pk_hbm.at[pl.ds(0, ECHUNK)], p, s).wait()
        pltpu.make_async_copy(ew_hbm.at[pl.ds(0, ECHUNK)], w, s).wait()

    def process(slot):
        p, w, _ = bufs[slot]

        @plsc.parallel_loop(0, ECHUNK, step=VL, unroll=5)
        def _vec(j):
            pp = p[pl.ds(j, VL)]
            ww = w[pl.ds(j, VL)]
            rr = lax.shift_right_logical(pp, jnp.int32(14))
            cc = lax.bitwise_and(pp, jnp.int32(0x3FFF))
            for f in range(FPT):
                vals = plsc.load_gather(ys[f], [rr])
                plsc.addupdate_scatter(accs[f], [cc], vals * ww)

    start(0, 0)
    start(1, 1)

    @pl.loop(0, nchunk, step=2)
    def _outer(i):
        wait(0)
        process(0)
        start(0, i + 2)
        wait(1)
        process(1)
        start(1, i + 3)

    wait(0)
    wait(1)
    for f in range(FPT):
        pltpu.sync_copy(accs[f], out_hbm.at[wid * FPT + f])


@jax.jit
def _agg(yT, pk, ew):
    k = pl.kernel(
        _agg_body,
        out_type=jax.ShapeDtypeStruct((D, N), jnp.float32),
        mesh=_vmesh,
        compiler_params=_sc_params,
        scratch_types=(
            [pltpu.VMEM((N,), jnp.float32)] * 8
            + [
                pltpu.VMEM((ECHUNK,), jnp.int32),
                pltpu.VMEM((ECHUNK,), jnp.float32),
                pltpu.VMEM((ECHUNK,), jnp.int32),
                pltpu.VMEM((ECHUNK,), jnp.float32),
                pltpu.SemaphoreType.DMA,
                pltpu.SemaphoreType.DMA,
            ]
        ),
    )
    return k(yT, pk, ew)


def _pack_body(ei_ref, o_ref):
    o_ref[...] = jnp.bitwise_or(
        jnp.left_shift(ei_ref[0:1, :], 14), ei_ref[1:2, :])


def _pack(ei):
    return pl.pallas_call(
        _pack_body,
        out_shape=jax.ShapeDtypeStruct((1, E), jnp.int32),
    )(ei).reshape(E)

def _mm_t_body(W_ref, x_ref, o_ref):
    o_ref[...] = lax.dot_general(
        W_ref[...], x_ref[...], (((0,), (1,)), ((), ())),
        preferred_element_type=jnp.float32)


def _mm_t(W, x):
    return pl.pallas_call(
        _mm_t_body,
        grid=(G,),
        in_specs=[
            pl.BlockSpec((D, D), lambda i: (0, 0)),
            pl.BlockSpec((R, D), lambda i: (i, 0)),
        ],
        out_specs=pl.BlockSpec((D, R), lambda i: (0, i)),
        out_shape=jax.ShapeDtypeStruct((D, N), jnp.float32),
    )(W, x)


def _dis_y_body(degp_ref, xwT_ref, dis_ref, yT_ref):
    deg = jnp.sum(degp_ref[...], axis=0, keepdims=True) + 1.0
    dis = jnp.where(deg > 0, lax.rsqrt(deg), 0.0)
    dis_ref[...] = dis
    yT_ref[...] = xwT_ref[...] * dis


def _dis_y(deg_part, xwT):
    return pl.pallas_call(
        _dis_y_body,
        grid=(G,),
        in_specs=[
            pl.BlockSpec((NW, R), lambda i: (0, i)),
            pl.BlockSpec((D, R), lambda i: (0, i)),
        ],
        out_specs=[
            pl.BlockSpec((1, R), lambda i: (0, i)),
            pl.BlockSpec((D, R), lambda i: (0, i)),
        ],
        out_shape=[
            jax.ShapeDtypeStruct((1, N), jnp.float32),
            jax.ShapeDtypeStruct((D, N), jnp.float32),
        ],
    )(deg_part, xwT)


def _post_mm_body(aggT_ref, yT_ref, dis_ref, b_ref, a_ref, W_ref, yT_out_ref):
    dis = dis_ref[...]
    h = dis * (aggT_ref[...] + yT_ref[...]) + b_ref[...]
    h = jnp.where(h >= 0, h, a_ref[...] * h)
    xwT = lax.dot_general(
        W_ref[...], h, (((0,), (0,)), ((), ())),
        preferred_element_type=jnp.float32)
    yT_out_ref[...] = xwT * dis


def _post_mm(aggT, yT, dis2d, b, a, W):
    return pl.pallas_call(
        _post_mm_body,
        grid=(G,),
        in_specs=[
            pl.BlockSpec((D, R), lambda i: (0, i)),
            pl.BlockSpec((D, R), lambda i: (0, i)),
            pl.BlockSpec((1, R), lambda i: (0, i)),
            pl.BlockSpec((D, 1), lambda i: (0, 0)),
            pl.BlockSpec((D, 1), lambda i: (0, 0)),
            pl.BlockSpec((D, D), lambda i: (0, 0)),
        ],
        out_specs=pl.BlockSpec((D, R), lambda i: (0, i)),
        out_shape=jax.ShapeDtypeStruct((D, N), jnp.float32),
    )(aggT, yT, dis2d, b, a, W)


def _final_body(aggT_ref, yT_ref, dis_ref, b_ref, a_ref, o_ref):
    h = dis_ref[...] * (aggT_ref[...] + yT_ref[...]) + b_ref[...]
    h = jnp.where(h >= 0, h, a_ref[...] * h)
    o_ref[...] = h.T


def _final(aggT, yT, dis2d, b, a):
    return pl.pallas_call(
        _final_body,
        grid=(G,),
        in_specs=[
            pl.BlockSpec((D, R), lambda i: (0, i)),
            pl.BlockSpec((D, R), lambda i: (0, i)),
            pl.BlockSpec((1, R), lambda i: (0, i)),
            pl.BlockSpec((D, 1), lambda i: (0, 0)),
            pl.BlockSpec((D, 1), lambda i: (0, 0)),
        ],
        out_specs=pl.BlockSpec((R, D), lambda i: (i, 0)),
        out_shape=jax.ShapeDtypeStruct((N, D), jnp.float32),
    )(aggT, yT, dis2d, b, a)


def kernel(x, edge_index, edge_weight, W1, b1, a1, W2, b2, a2, W3, b3, a3):
    col = edge_index[1]
    ew = edge_weight

    pk = _pack(edge_index)
    deg_part = _deg_partials(col, ew)
    xw1T = _mm_t(W1, x)
    dis2d, y1T = _dis_y(deg_part, xw1T)

    b1c, a1c = b1.reshape(D, 1), a1.reshape(D, 1)
    b2c, a2c = b2.reshape(D, 1), a2.reshape(D, 1)
    b3c, a3c = b3.reshape(D, 1), a3.reshape(D, 1)

    agg1T = _agg(y1T, pk, ew)
    y2T = _post_mm(agg1T, y1T, dis2d, b1c, a1c, W2)
    agg2T = _agg(y2T, pk, ew)
    y3T = _post_mm(agg2T, y2T, dis2d, b2c, a2c, W3)
    agg3T = _agg(y3T, pk, ew)
    return _final(agg3T, y3T, dis2d, b3c, a3c)


# speedup vs baseline: 1.2301x; 1.0988x over previous
import dataclasses

import jax
import jax.numpy as jnp
from jax import lax
from jax.experimental import pallas as pl
from jax.experimental.pallas import tpu as pltpu
from jax.experimental.pallas import tpu_sc as plsc

N = 10000
E = 320000
D = 128
NC = 2
NS = 16
NW = NC * NS
FPT = D // NW
VL = 16
ECHUNK = 8000
DCHUNK = 2000
R = N
G = N // R

_vmesh = plsc.VectorSubcoreMesh(core_axis_name="c", subcore_axis_name="s")

_sc_params = pltpu.CompilerParams()
if "needs_layout_passes" in pltpu.CompilerParams.__dataclass_fields__:
    _sc_params = dataclasses.replace(_sc_params, needs_layout_passes=False)


def _deg_body(col_hbm, ew_hbm, out_hbm, col_v, ew_v, deg_v):
    wid = lax.axis_index("s") * NC + lax.axis_index("c")

    @pl.loop(0, N, step=VL)
    def _zero(i):
        deg_v[pl.ds(i, VL)] = jnp.zeros((VL,), jnp.float32)

    epw = E // NW
    base = wid * epw

    @pl.loop(0, epw, step=DCHUNK)
    def _chunk(i):
        pltpu.sync_copy(col_hbm.at[pl.ds(base + i, DCHUNK)], col_v)
        pltpu.sync_copy(ew_hbm.at[pl.ds(base + i, DCHUNK)], ew_v)

        @plsc.parallel_loop(0, DCHUNK, step=VL, unroll=4)
        def _vec(j):
            c = col_v[pl.ds(j, VL)]
            w = ew_v[pl.ds(j, VL)]
            plsc.addupdate_scatter(deg_v, [c], w)

    pltpu.sync_copy(deg_v, out_hbm.at[wid])


@jax.jit
def _deg_partials(col, ew):
    k = pl.kernel(
        _deg_body,
        out_type=jax.ShapeDtypeStruct((NW, N), jnp.float32),
        mesh=_vmesh,
        compiler_params=_sc_params,
        scratch_types=[
            pltpu.VMEM((DCHUNK,), jnp.int32),
            pltpu.VMEM((DCHUNK,), jnp.float32),
            pltpu.VMEM((N,), jnp.float32),
        ],
    )
    return k(col, ew)


def _agg_body(ypT_hbm, pk_hbm, ew_hbm, out_hbm,
              yp0, yp1, a00, a01, a10, a11,
              pk0, ew0, pk1, ew1, sem0, sem1):
    wid = lax.axis_index("s") * NC + lax.axis_index("c")
    yps = (yp0, yp1)
    accs = (a00, a01, a10, a11)
    for g in range(2):
        pltpu.sync_copy(ypT_hbm.at[wid * 2 + g], yps[g])
    for f in range(4):
        @plsc.parallel_loop(0, N, step=VL, unroll=8)
        def _zero(i, f=f):
            accs[f][pl.ds(i, VL)] = jnp.zeros((VL,), jnp.float32)

    bufs = ((pk0, ew0, sem0), (pk1, ew1, sem1))
    nchunk = E // ECHUNK

    def start(slot, ci):
        p, w, s = bufs[slot]
        off = jnp.minimum(ci * ECHUNK, E - ECHUNK)
        pltpu.async_copy(pk_hbm.at[pl.ds(off, ECHUNK)], p, s)
        pltpu.async_copy(ew_hbm.at[pl.ds(off, ECHUNK)], w, s)

    def wait(slot):
        p, w, s = bufs[slot]
        pltpu.make_async_copy(pk_hbm.at[pl.ds(0, ECHUNK)], p, s).wait()
        pltpu.make_async_copy(ew_hbm.at[pl.ds(0, ECHUNK)], w, s).wait()

    def process(slot):
        p, w, _ = bufs[slot]

        @plsc.parallel_loop(0, ECHUNK, step=VL, unroll=5)
        def _vec(j):
            pp = p[pl.ds(j, VL)]
            ww = w[pl.ds(j, VL)]
            rr = lax.shift_right_logical(pp, jnp.int32(14))
            cc = lax.bitwise_and(pp, jnp.int32(0x3FFF))
            for g in range(2):
                wv = plsc.load_gather(yps[g], [rr])
                va = plsc.bitcast(lax.shift_left(wv, jnp.int32(16)),
                                  jnp.float32)
                vb = plsc.bitcast(lax.bitwise_and(wv, jnp.int32(-65536)),
                                  jnp.float32)
                plsc.addupdate_scatter(accs[2 * g], [cc], va * ww)
                plsc.addupdate_scatter(accs[2 * g + 1], [cc], vb * ww)

    start(0, 0)
    start(1, 1)

    @pl.loop(0, nchunk, step=2)
    def _outer(i):
        wait(0)
        process(0)
        start(0, i + 2)
        wait(1)
        process(1)
        start(1, i + 3)

    wait(0)
    wait(1)
    for g in range(2):
        pltpu.sync_copy(accs[2 * g], out_hbm.at[wid * 2 + g])
        pltpu.sync_copy(accs[2 * g + 1], out_hbm.at[wid * 2 + g + 64])


@jax.jit
def _agg(ypT, pk, ew):
    k = pl.kernel(
        _agg_body,
        out_type=jax.ShapeDtypeStruct((D, N), jnp.float32),
        mesh=_vmesh,
        compiler_params=_sc_params,
        scratch_types=(
            [pltpu.VMEM((N,), jnp.int32)] * 2
            + [pltpu.VMEM((N,), jnp.float32)] * 4
            + [
                pltpu.VMEM((ECHUNK,), jnp.int32),
                pltpu.VMEM((ECHUNK,), jnp.float32),
                pltpu.VMEM((ECHUNK,), jnp.int32),
                pltpu.VMEM((ECHUNK,), jnp.float32),
                pltpu.SemaphoreType.DMA,
                pltpu.SemaphoreType.DMA,
            ]
        ),
    )
    return k(ypT, pk, ew)


def _pack_body(ei_ref, o_ref):
    o_ref[...] = jnp.bitwise_or(
        jnp.left_shift(ei_ref[0:1, :], 14), ei_ref[1:2, :])


def _pack(ei):
    return pl.pallas_call(
        _pack_body,
        out_shape=jax.ShapeDtypeStruct((1, E), jnp.int32),
    )(ei).reshape(E)

def _mm_t_body(W_ref, x_ref, o_ref):
    o_ref[...] = lax.dot_general(
        W_ref[...], x_ref[...], (((0,), (1,)), ((), ())),
        preferred_element_type=jnp.float32)


def _mm_t(W, x):
    return pl.pallas_call(
        _mm_t_body,
        grid=(G,),
        in_specs=[
            pl.BlockSpec((D, D), lambda i: (0, 0)),
            pl.BlockSpec((R, D), lambda i: (i, 0)),
        ],
        out_specs=pl.BlockSpec((D, R), lambda i: (0, i)),
        out_shape=jax.ShapeDtypeStruct((D, N), jnp.float32),
    )(W, x)


def _packy(yT32):
    u = lax.bitcast_convert_type(yT32.astype(jnp.bfloat16), jnp.uint16)
    lo = u[:64].astype(jnp.uint32)
    hi = u[64:].astype(jnp.uint32)
    return lax.bitcast_convert_type(lo | (hi << 16), jnp.int32)


def _dis_y_body(degp_ref, xwT_ref, dis_ref, yT_ref, ypT_ref):
    deg = jnp.sum(degp_ref[...], axis=0, keepdims=True) + 1.0
    dis = jnp.where(deg > 0, lax.rsqrt(deg), 0.0)
    dis_ref[...] = dis
    yT = xwT_ref[...] * dis
    yT_ref[...] = yT
    ypT_ref[...] = _packy(yT)


def _dis_y(deg_part, xwT):
    return pl.pallas_call(
        _dis_y_body,
        grid=(G,),
        in_specs=[
            pl.BlockSpec((NW, R), lambda i: (0, i)),
            pl.BlockSpec((D, R), lambda i: (0, i)),
        ],
        out_specs=[
            pl.BlockSpec((1, R), lambda i: (0, i)),
            pl.BlockSpec((D, R), lambda i: (0, i)),
            pl.BlockSpec((D // 2, R), lambda i: (0, i)),
        ],
        out_shape=[
            jax.ShapeDtypeStruct((1, N), jnp.float32),
            jax.ShapeDtypeStruct((D, N), jnp.float32),
            jax.ShapeDtypeStruct((D // 2, N), jnp.int32),
        ],
    )(deg_part, xwT)


def _post_mm_body(aggT_ref, yT_ref, dis_ref, b_ref, a_ref, W_ref,
                  yT_out_ref, ypT_out_ref):
    dis = dis_ref[...]
    h = dis * (aggT_ref[...] + yT_ref[...]) + b_ref[...]
    h = jnp.where(h >= 0, h, a_ref[...] * h)
    xwT = lax.dot_general(
        W_ref[...], h, (((0,), (0,)), ((), ())),
        preferred_element_type=jnp.float32)
    yT = xwT * dis
    yT_out_ref[...] = yT
    ypT_out_ref[...] = _packy(yT)


def _post_mm(aggT, yT, dis2d, b, a, W):
    return pl.pallas_call(
        _post_mm_body,
        grid=(G,),
        in_specs=[
            pl.BlockSpec((D, R), lambda i: (0, i)),
            pl.BlockSpec((D, R), lambda i: (0, i)),
            pl.BlockSpec((1, R), lambda i: (0, i)),
            pl.BlockSpec((D, 1), lambda i: (0, 0)),
            pl.BlockSpec((D, 1), lambda i: (0, 0)),
            pl.BlockSpec((D, D), lambda i: (0, 0)),
        ],
        out_specs=[
            pl.BlockSpec((D, R), lambda i: (0, i)),
            pl.BlockSpec((D // 2, R), lambda i: (0, i)),
        ],
        out_shape=[
            jax.ShapeDtypeStruct((D, N), jnp.float32),
            jax.ShapeDtypeStruct((D // 2, N), jnp.int32),
        ],
    )(aggT, yT, dis2d, b, a, W)


def _final_body(aggT_ref, yT_ref, dis_ref, b_ref, a_ref, o_ref):
    h = dis_ref[...] * (aggT_ref[...] + yT_ref[...]) + b_ref[...]
    h = jnp.where(h >= 0, h, a_ref[...] * h)
    o_ref[...] = h.T


def _final(aggT, yT, dis2d, b, a):
    return pl.pallas_call(
        _final_body,
        grid=(G,),
        in_specs=[
            pl.BlockSpec((D, R), lambda i: (0, i)),
            pl.BlockSpec((D, R), lambda i: (0, i)),
            pl.BlockSpec((1, R), lambda i: (0, i)),
            pl.BlockSpec((D, 1), lambda i: (0, 0)),
            pl.BlockSpec((D, 1), lambda i: (0, 0)),
        ],
        out_specs=pl.BlockSpec((R, D), lambda i: (i, 0)),
        out_shape=jax.ShapeDtypeStruct((N, D), jnp.float32),
    )(aggT, yT, dis2d, b, a)


def kernel(x, edge_index, edge_weight, W1, b1, a1, W2, b2, a2, W3, b3, a3):
    col = edge_index[1]
    ew = edge_weight

    pk = _pack(edge_index)
    deg_part = _deg_partials(col, ew)
    xw1T = _mm_t(W1, x)
    dis2d, y1T, y1p = _dis_y(deg_part, xw1T)

    b1c, a1c = b1.reshape(D, 1), a1.reshape(D, 1)
    b2c, a2c = b2.reshape(D, 1), a2.reshape(D, 1)
    b3c, a3c = b3.reshape(D, 1), a3.reshape(D, 1)

    agg1T = _agg(y1p, pk, ew)
    y2T, y2p = _post_mm(agg1T, y1T, dis2d, b1c, a1c, W2)
    agg2T = _agg(y2p, pk, ew)
    y3T, y3p = _post_mm(agg2T, y2T, dis2d, b2c, a2c, W3)
    agg3T = _agg(y3p, pk, ew)
    return _final(agg3T, y3T, dis2d, b3c, a3c)
